# Initial kernel scaffold; baseline (speedup 1.0000x reference)
import jax
import jax.numpy as jnp
from jax.experimental import pallas as pl


def kernel(data, wT, knots_x, knots_y, knots_d):
    raise NotImplementedError("write your pallas kernel here")


# speedup vs baseline: 4496.0203x; 4496.0203x over previous
import functools

import jax
import jax.numpy as jnp
from jax.experimental import pallas as pl
from jax.experimental.pallas import tpu as pltpu

H, W, C = 64, 64, 3
KH, KW = 4, 4
NH, NW = H // KH, W // KW
NPATCH = NH * NW * C
PDIM = KH * KW
NCOMP = 16
NBIN = 200
NPAD = 256

BT = 128
BP = BT // NCOMP
NBLK = NPATCH // BP


def _gather(tab, idx):
    m = jnp.bitwise_and(idx, 127)
    lo = jnp.take_along_axis(tab[:, :128], m, axis=1)
    hi = jnp.take_along_axis(tab[:, 128:], m, axis=1)
    return jnp.where(idx >= 128, hi, lo)


def _spline_body(x_ref, wb_ref, wbt_ref, kx_ref, ky_ref, kd_ref,
                 out_ref, lj_ref):
    j = pl.program_id(1)
    x0 = x_ref[...]
    wf = wbt_ref[0]
    wb = wb_ref[0]
    kx = kx_ref[...]
    ky = ky_ref[...]
    kd = kd_ref[...]

    x = jnp.dot(wf, x0, preferred_element_type=jnp.float32)

    c = jnp.zeros(x.shape, jnp.int32)
    for s in (128, 64, 32, 16, 8, 4, 2, 1):
        v = _gather(kx, c + (s - 1))
        c = jnp.where(v < x, c + s, c)

    k = jnp.clip(c - 1, 0, NBIN - 2)
    k1 = k + 1
    xk, xk1 = _gather(kx, k), _gather(kx, k1)
    yk, yk1 = _gather(ky, k), _gather(ky, k1)
    dk, dk1 = _gather(kd, k), _gather(kd, k1)

    wx = xk1 - xk
    dy = yk1 - yk
    inv_wx = 1.0 / wx
    sl = dy * inv_wx
    xi = jnp.clip((x - xk) * inv_wx, 0.0, 1.0)
    xi1 = 1.0 - xi
    xx1 = xi * xi1
    den = sl + (dk + dk1 - 2.0 * sl) * xx1
    inv_den = 1.0 / den
    y_in = yk + dy * (sl * xi * xi + dk * xx1) * inv_den
    dydx_in = sl * sl * (dk1 * xi * xi + 2.0 * sl * xx1 + dk * xi1 * xi1) \
        * inv_den * inv_den

    below = c < 1
    above = c >= NBIN
    y = jnp.where(below, yk + (x - xk) * dk,
                  jnp.where(above, yk1 + (x - xk1) * dk1, y_in))
    dydx = jnp.where(below, dk, jnp.where(above, dk1, dydx_in))

    out_ref[...] = x0 + jnp.dot(wb, y - x, preferred_element_type=jnp.float32)

    part = jnp.sum(jnp.log(dydx), axis=0, keepdims=True)

    @pl.when(j == 0)
    def _():
        lj_ref[...] = part

    @pl.when(j != 0)
    def _():
        lj_ref[...] = lj_ref[...] + part


@functools.partial(jax.jit, static_argnames=("interpret",))
def kernel(data, wT, knots_x, knots_y, knots_d, interpret=False):
    n = data.shape[0]
    t = NPATCH * NCOMP
    bn = min(2048, n)

    xpt = (data.reshape(n, NH, KH, NW, KW, C)
           .transpose(1, 3, 5, 2, 4, 0).reshape(t, n))

    w4 = wT.reshape(NBLK, BP, PDIM, NCOMP)
    eye = jnp.eye(BP, dtype=wT.dtype)
    wblk = jnp.einsum('gpkc,pq->gpkqc', w4, eye).reshape(NBLK, BT, BT)
    wblk_t = wblk.transpose(0, 2, 1)

    pad = jnp.full((t, NPAD - NBIN), jnp.inf, jnp.float32)
    kxp = jnp.concatenate([knots_x, pad], axis=1)
    kyp = jnp.concatenate([knots_y, pad], axis=1)
    kdp = jnp.concatenate([knots_d, pad], axis=1)

    grid = (n // bn, NBLK)
    out_t, logj2 = pl.pallas_call(
        _spline_body,
        grid=grid,
        in_specs=[
            pl.BlockSpec((BT, bn), lambda i, j: (j, i)),
            pl.BlockSpec((1, BT, BT), lambda i, j: (j, 0, 0)),
            pl.BlockSpec((1, BT, BT), lambda i, j: (j, 0, 0)),
            pl.BlockSpec((BT, NPAD), lambda i, j: (j, 0)),
            pl.BlockSpec((BT, NPAD), lambda i, j: (j, 0)),
            pl.BlockSpec((BT, NPAD), lambda i, j: (j, 0)),
        ],
        out_specs=[
            pl.BlockSpec((BT, bn), lambda i, j: (j, i)),
            pl.BlockSpec((1, bn), lambda i, j: (0, i)),
        ],
        out_shape=[
            jax.ShapeDtypeStruct((t, n), jnp.float32),
            jax.ShapeDtypeStruct((1, n), jnp.float32),
        ],
        compiler_params=pltpu.CompilerParams(
            dimension_semantics=("parallel", "arbitrary"),
            vmem_limit_bytes=100 * 1024 * 1024,
        ),
        interpret=interpret,
    )(xpt, wblk, wblk_t, kxp, kyp, kdp)

    out = (out_t.reshape(NH, NW, C, KH, KW, n)
           .transpose(5, 0, 3, 1, 4, 2).reshape(n, H * W * C))
    return out, logj2[0]


# speedup vs baseline: 4959.5722x; 1.1031x over previous
import functools

import jax
import jax.numpy as jnp
from jax.experimental import pallas as pl
from jax.experimental.pallas import tpu as pltpu

H, W, C = 64, 64, 3
KH, KW = 4, 4
NH, NW = H // KH, W // KW
NPATCH = NH * NW * C
PDIM = KH * KW
NCOMP = 16
NBIN = 200
NPAD = 256

BT = 128
BP = BT // NCOMP
NBLK = NPATCH // BP


def _gather(tab, idx):
    m = jnp.bitwise_and(idx, 127)
    lo = jnp.take_along_axis(tab[:, :128], m, axis=1)
    hi = jnp.take_along_axis(tab[:, 128:], m, axis=1)
    return jnp.where(idx >= 128, hi, lo)


def _spline_body(x_ref, wb_ref, wbt_ref, kx_ref, ky_ref, kd_ref,
                 out_ref, lj_ref):
    j = pl.program_id(1)
    x0 = x_ref[...]
    wf = wbt_ref[0]
    wb = wb_ref[0]
    kx = kx_ref[...]
    ky = ky_ref[...]
    kd = kd_ref[...]

    x = jnp.dot(wf, x0, preferred_element_type=jnp.float32)

    col = lambda b: kx[:, b:b + 1]
    c = jnp.where(col(127) < x, 128, 0)
    v64 = jnp.where(c == 128, col(191), col(63))
    c = jnp.where(v64 < x, c + 64, c)
    v32hi = jnp.where(c >= 192, col(223), col(159))
    v32lo = jnp.where(c >= 64, col(95), col(31))
    v32 = jnp.where(c >= 128, v32hi, v32lo)
    c = jnp.where(v32 < x, c + 32, c)
    for s in (16, 8, 4, 2, 1):
        v = _gather(kx, c + (s - 1))
        c = jnp.where(v < x, c + s, c)

    k = jnp.clip(c - 1, 0, NBIN - 2)
    k1 = k + 1
    xk, xk1 = _gather(kx, k), _gather(kx, k1)
    yk, yk1 = _gather(ky, k), _gather(ky, k1)
    dk, dk1 = _gather(kd, k), _gather(kd, k1)

    wx = xk1 - xk
    dy = yk1 - yk
    inv_wx = 1.0 / wx
    sl = dy * inv_wx
    xi = jnp.clip((x - xk) * inv_wx, 0.0, 1.0)
    xi1 = 1.0 - xi
    xx1 = xi * xi1
    den = sl + (dk + dk1 - 2.0 * sl) * xx1
    inv_den = 1.0 / den
    y_in = yk + dy * (sl * xi * xi + dk * xx1) * inv_den
    dydx_in = sl * sl * (dk1 * xi * xi + 2.0 * sl * xx1 + dk * xi1 * xi1) \
        * inv_den * inv_den

    below = c < 1
    above = c >= NBIN
    y = jnp.where(below, yk + (x - xk) * dk,
                  jnp.where(above, yk1 + (x - xk1) * dk1, y_in))
    dydx = jnp.where(below, dk, jnp.where(above, dk1, dydx_in))

    out_ref[...] = x0 + jnp.dot(wb, y - x, preferred_element_type=jnp.float32)

    part = jnp.sum(jnp.log(dydx), axis=0, keepdims=True)

    @pl.when(j == 0)
    def _():
        lj_ref[...] = part

    @pl.when(j != 0)
    def _():
        lj_ref[...] = lj_ref[...] + part


@functools.partial(jax.jit, static_argnames=("interpret",))
def kernel(data, wT, knots_x, knots_y, knots_d, interpret=False):
    n = data.shape[0]
    t = NPATCH * NCOMP
    bn = min(2048, n)

    xpt = (data.reshape(n, NH, KH, NW, KW, C)
           .transpose(1, 3, 5, 2, 4, 0).reshape(t, n))

    w4 = wT.reshape(NBLK, BP, PDIM, NCOMP)
    eye = jnp.eye(BP, dtype=wT.dtype)
    wblk = jnp.einsum('gpkc,pq->gpkqc', w4, eye).reshape(NBLK, BT, BT)
    wblk_t = wblk.transpose(0, 2, 1)

    pad = jnp.full((t, NPAD - NBIN), jnp.inf, jnp.float32)
    kxp = jnp.concatenate([knots_x, pad], axis=1)
    kyp = jnp.concatenate([knots_y, pad], axis=1)
    kdp = jnp.concatenate([knots_d, pad], axis=1)

    grid = (n // bn, NBLK)
    out_t, logj2 = pl.pallas_call(
        _spline_body,
        grid=grid,
        in_specs=[
            pl.BlockSpec((BT, bn), lambda i, j: (j, i)),
            pl.BlockSpec((1, BT, BT), lambda i, j: (j, 0, 0)),
            pl.BlockSpec((1, BT, BT), lambda i, j: (j, 0, 0)),
            pl.BlockSpec((BT, NPAD), lambda i, j: (j, 0)),
            pl.BlockSpec((BT, NPAD), lambda i, j: (j, 0)),
            pl.BlockSpec((BT, NPAD), lambda i, j: (j, 0)),
        ],
        out_specs=[
            pl.BlockSpec((BT, bn), lambda i, j: (j, i)),
            pl.BlockSpec((1, bn), lambda i, j: (0, i)),
        ],
        out_shape=[
            jax.ShapeDtypeStruct((t, n), jnp.float32),
            jax.ShapeDtypeStruct((1, n), jnp.float32),
        ],
        compiler_params=pltpu.CompilerParams(
            dimension_semantics=("parallel", "arbitrary"),
            vmem_limit_bytes=100 * 1024 * 1024,
        ),
        interpret=interpret,
    )(xpt, wblk, wblk_t, kxp, kyp, kdp)

    out = (out_t.reshape(NH, NW, C, KH, KW, n)
           .transpose(5, 0, 3, 1, 4, 2).reshape(n, H * W * C))
    return out, logj2[0]
